# baseline (speedup 1.0000x reference)
import functools

import jax
import jax.numpy as jnp
from jax import lax
from jax.experimental import pallas as pl
from jax.experimental.pallas import tpu as pltpu
from jax.experimental.pallas import tpu_sc as plsc

EPS = 1e-15
N = 10000
E = 320000
D = 128

NC = 2
NS = 16
NW = NC * NS
CHUNK = 128
GCHUNK = 8
GROUPS = 10
EDGES_PER_TILE = GROUPS * GCHUNK * CHUNK
E_PAD = NW * EDGES_PER_TILE
N_ACC = 10240
ROWS_PER_TILE = N_ACC // NS
ZROWS = 128

BLK = 2000


def _artanh(x):
    x = jnp.clip(x, -1.0 + 1e-5, 1.0 - 1e-5)
    return 0.5 * (jnp.log1p(x) - jnp.log1p(-x))


def _rownorm(x):
    return jnp.clip(jnp.sqrt(jnp.sum(x * x, axis=-1, keepdims=True)), EPS, None)


def _dense_body(x_ref, w_ref, b_ref, c_ref, o_ref):
    c = c_ref[...]
    sqrt_c = jnp.sqrt(c)
    x = x_ref[...]
    x_norm = _rownorm(x)
    mx = jnp.dot(x, w_ref[...], preferred_element_type=jnp.float32)
    mx_norm = _rownorm(mx)
    res = jnp.tanh(mx_norm / x_norm * _artanh(sqrt_c * x_norm)) * mx / (mx_norm * sqrt_c)
    zero_mask = jnp.max(jnp.abs(mx), axis=-1, keepdims=True) == 0.0
    out = jnp.where(zero_mask, 0.0, res)

    bias = b_ref[...]
    bn = _rownorm(bias)
    hb = jnp.tanh(sqrt_c * bn) * bias / (sqrt_c * bn)
    hbn = _rownorm(hb)
    maxnorm = (1.0 - 4e-3) / sqrt_c
    hb = jnp.where(hbn > maxnorm, hb / hbn * maxnorm, hb)

    x2 = jnp.sum(out * out, axis=-1, keepdims=True)
    y2 = jnp.sum(hb * hb, axis=-1, keepdims=True)
    xy = jnp.sum(out * hb, axis=-1, keepdims=True)
    num = (1.0 + 2.0 * c * xy + c * y2) * out + (1.0 - c * x2) * hb
    den = 1.0 + 2.0 * c * xy + c * c * x2 * y2
    oa = num / jnp.clip(den, EPS, None)

    on = _rownorm(oa)
    o_ref[...] = _artanh(sqrt_c * on) * oa / (sqrt_c * on)


def _dense_stage(x, w, b2, c2):
    return pl.pallas_call(
        _dense_body,
        grid=(N // BLK,),
        in_specs=[
            pl.BlockSpec((BLK, D), lambda i: (i, 0)),
            pl.BlockSpec((D, D), lambda i: (0, 0)),
            pl.BlockSpec((1, D), lambda i: (0, 0)),
            pl.BlockSpec((1, 1), lambda i: (0, 0)),
        ],
        out_specs=pl.BlockSpec((BLK, D), lambda i: (i, 0)),
        out_shape=jax.ShapeDtypeStruct((N, D), jnp.float32),
    )(x, w, b2, c2)


def _final_body(p_ref, c_ref, o_ref):
    c = c_ref[...]
    sqrt_c = jnp.sqrt(c)
    p = p_ref[...]
    s = p[:, :D] + p[:, D:]
    un = _rownorm(s)
    em = jnp.tanh(sqrt_c * un) * s / (sqrt_c * un)
    en = _rownorm(em)
    maxnorm = (1.0 - 4e-3) / sqrt_c
    o_ref[...] = jnp.where(en > maxnorm, em / en * maxnorm, em)


def _final_stage(parts, c2):
    return pl.pallas_call(
        _final_body,
        grid=(N // BLK,),
        in_specs=[
            pl.BlockSpec((BLK, NC * D), lambda i: (i, 0)),
            pl.BlockSpec((1, 1), lambda i: (0, 0)),
        ],
        out_specs=pl.BlockSpec((BLK, D), lambda i: (i, 0)),
        out_shape=jax.ShapeDtypeStruct((N, D), jnp.float32),
    )(parts, c2)


def _bcast_lane(vec, lane):
    idx = jnp.full((16, 1), lane, jnp.int32)
    dnums = lax.GatherDimensionNumbers(
        offset_dims=(), collapsed_slice_dims=(0,), start_index_map=(0,))
    return lax.gather(vec, idx, dnums, (1,),
                      mode=lax.GatherScatterMode.PROMISE_IN_BOUNDS)


def _sc_body(x_hbm, src_hbm, dst_hbm, val_hbm, out_hbm,
             acc, srcb, dstb, valb, rows, gsem, ssem):
    cid = lax.axis_index("c")
    sid = lax.axis_index("s")
    wid = cid * NS + sid

    def _zrow(i, carry):
        for q in range(D // 16):
            rows[0, i, pl.ds(q * 16, 16)] = jnp.zeros((16,), jnp.float32)
        return carry
    lax.fori_loop(0, CHUNK, _zrow, 0)
    for k in range(ROWS_PER_TILE // CHUNK):
        pltpu.sync_copy(rows.at[0],
                        acc.at[pl.ds(sid * ROWS_PER_TILE + k * CHUNK, CHUNK)])

    plsc.subcore_barrier()

    def _group(og, carry):
        pltpu.sync_copy(src_hbm.at[wid, og], srcb)
        pltpu.sync_copy(dst_hbm.at[wid, og], dstb)
        pltpu.sync_copy(val_hbm.at[wid, og], valb)
        pltpu.async_copy(x_hbm.at[srcb.at[0]], rows.at[0], gsem.at[0])

        def _chunk(j, carry2):
            p = j % 2
            np_ = 1 - p
            @pl.when(j + 1 < GCHUNK)
            def _start_next():
                pltpu.async_copy(x_hbm.at[srcb.at[j + 1]], rows.at[np_],
                                 gsem.at[np_])
            pltpu.make_async_copy(x_hbm.at[srcb.at[j]], rows.at[p],
                                  gsem.at[p]).wait()
            for g in range(CHUNK // 16):
                vv = valb[j, pl.ds(g * 16, 16)]
                for l in range(16):
                    e = g * 16 + l
                    vb = _bcast_lane(vv, l)
                    for q in range(D // 16):
                        rows[p, e, pl.ds(q * 16, 16)] = (
                            rows[p, e, pl.ds(q * 16, 16)] * vb)
            pltpu.async_copy(rows.at[p], acc.at[dstb.at[j]], ssem.at[p],
                             add=True)
            @pl.when(j > 0)
            def _wait_prev():
                pltpu.make_async_copy(rows.at[np_], acc.at[dstb.at[j - 1]],
                                      ssem.at[np_]).wait()
            return carry2
        lax.fori_loop(0, GCHUNK, _chunk, 0)
        lastp = (GCHUNK - 1) % 2
        pltpu.make_async_copy(rows.at[lastp], acc.at[dstb.at[GCHUNK - 1]],
                              ssem.at[lastp]).wait()
        return carry
    lax.fori_loop(0, GROUPS, _group, 0)

    plsc.subcore_barrier()
    for k in range(ROWS_PER_TILE // ZROWS):
        r0 = sid * ROWS_PER_TILE + k * ZROWS
        pltpu.sync_copy(acc.at[pl.ds(r0, ZROWS)],
                        out_hbm.at[pl.ds(r0, ZROWS), pl.ds(cid * D, D)])


@functools.cache
def _sc_agg():
    return pl.kernel(
        _sc_body,
        out_type=jax.ShapeDtypeStruct((N_ACC, NC * D), jnp.float32),
        mesh=plsc.VectorSubcoreMesh(core_axis_name="c", subcore_axis_name="s",
                                    num_cores=NC, num_subcores=NS),
        scratch_types=[
            pltpu.VMEM_SHARED((N_ACC, D), jnp.float32),
            pltpu.VMEM((GCHUNK, CHUNK), jnp.int32),
            pltpu.VMEM((GCHUNK, CHUNK), jnp.int32),
            pltpu.VMEM((GCHUNK, CHUNK), jnp.float32),
            pltpu.VMEM((2, CHUNK, D), jnp.float32),
            pltpu.SemaphoreType.DMA((2,)),
            pltpu.SemaphoreType.DMA((2,)),
        ],
    )


def kernel(input, adj_edge_index, adj_values, W, b, self_c):
    c2 = self_c.reshape(1, 1)
    b2 = b.reshape(1, D)
    h = _dense_stage(input, W, b2, c2)
    pad = E_PAD - E
    src = jnp.pad(adj_edge_index[1], (0, pad)).reshape(NW, GROUPS, GCHUNK, CHUNK)
    dst = jnp.pad(adj_edge_index[0], (0, pad)).reshape(NW, GROUPS, GCHUNK, CHUNK)
    val = jnp.pad(adj_values, (0, pad)).reshape(NW, GROUPS, GCHUNK, CHUNK)
    parts = _sc_agg()(h, src, dst, val)
    return _final_stage(parts, c2)


# speedup vs baseline: 2.6918x; 2.6918x over previous
import functools

import jax
import jax.numpy as jnp
from jax import lax
from jax.experimental import pallas as pl
from jax.experimental.pallas import tpu as pltpu
from jax.experimental.pallas import tpu_sc as plsc

EPS = 1e-15
N = 10000
E = 320000
D = 128

NC = 2
NS = 16
NW = NC * NS
CHUNK = 80
GCHUNK = 25
GROUPS = 5
EDGES_PER_TILE = GROUPS * GCHUNK * CHUNK
E_PAD = NW * EDGES_PER_TILE
N_ACC = 10240
ROWS_PER_TILE = N_ACC // NS
ZROWS = 128

BLK = 2000


def _artanh(x):
    x = jnp.clip(x, -1.0 + 1e-5, 1.0 - 1e-5)
    return 0.5 * (jnp.log1p(x) - jnp.log1p(-x))


def _rownorm(x):
    return jnp.clip(jnp.sqrt(jnp.sum(x * x, axis=-1, keepdims=True)), EPS, None)


def _dense_body(x_ref, w_ref, b_ref, c_ref, o_ref):
    c = c_ref[...]
    sqrt_c = jnp.sqrt(c)
    x = x_ref[...]
    x_norm = _rownorm(x)
    mx = jnp.dot(x, w_ref[...], preferred_element_type=jnp.float32)
    mx_norm = _rownorm(mx)
    res = jnp.tanh(mx_norm / x_norm * _artanh(sqrt_c * x_norm)) * mx / (mx_norm * sqrt_c)
    zero_mask = jnp.max(jnp.abs(mx), axis=-1, keepdims=True) == 0.0
    out = jnp.where(zero_mask, 0.0, res)

    bias = b_ref[...]
    bn = _rownorm(bias)
    hb = jnp.tanh(sqrt_c * bn) * bias / (sqrt_c * bn)
    hbn = _rownorm(hb)
    maxnorm = (1.0 - 4e-3) / sqrt_c
    hb = jnp.where(hbn > maxnorm, hb / hbn * maxnorm, hb)

    x2 = jnp.sum(out * out, axis=-1, keepdims=True)
    y2 = jnp.sum(hb * hb, axis=-1, keepdims=True)
    xy = jnp.sum(out * hb, axis=-1, keepdims=True)
    num = (1.0 + 2.0 * c * xy + c * y2) * out + (1.0 - c * x2) * hb
    den = 1.0 + 2.0 * c * xy + c * c * x2 * y2
    oa = num / jnp.clip(den, EPS, None)

    on = _rownorm(oa)
    o_ref[...] = _artanh(sqrt_c * on) * oa / (sqrt_c * on)


def _dense_stage(x, w, b2, c2):
    return pl.pallas_call(
        _dense_body,
        grid=(N // BLK,),
        in_specs=[
            pl.BlockSpec((BLK, D), lambda i: (i, 0)),
            pl.BlockSpec((D, D), lambda i: (0, 0)),
            pl.BlockSpec((1, D), lambda i: (0, 0)),
            pl.BlockSpec((1, 1), lambda i: (0, 0)),
        ],
        out_specs=pl.BlockSpec((BLK, D), lambda i: (i, 0)),
        out_shape=jax.ShapeDtypeStruct((N, D), jnp.float32),
    )(x, w, b2, c2)


def _final_body(p_ref, c_ref, o_ref):
    c = c_ref[...]
    sqrt_c = jnp.sqrt(c)
    s = p_ref[0] + p_ref[1]
    un = _rownorm(s)
    em = jnp.tanh(sqrt_c * un) * s / (sqrt_c * un)
    en = _rownorm(em)
    maxnorm = (1.0 - 4e-3) / sqrt_c
    o_ref[...] = jnp.where(en > maxnorm, em / en * maxnorm, em)


def _final_stage(parts, c2):
    return pl.pallas_call(
        _final_body,
        grid=(N // BLK,),
        in_specs=[
            pl.BlockSpec((NC, BLK, D), lambda i: (0, i, 0)),
            pl.BlockSpec((1, 1), lambda i: (0, 0)),
        ],
        out_specs=pl.BlockSpec((BLK, D), lambda i: (i, 0)),
        out_shape=jax.ShapeDtypeStruct((N, D), jnp.float32),
    )(parts, c2)


def _bcast_lane(vec, lane):
    idx = jnp.full((16, 1), lane, jnp.int32)
    dnums = lax.GatherDimensionNumbers(
        offset_dims=(), collapsed_slice_dims=(0,), start_index_map=(0,))
    return lax.gather(vec, idx, dnums, (1,),
                      mode=lax.GatherScatterMode.PROMISE_IN_BOUNDS)


def _sc_body(x_hbm, src_hbm, dst_hbm, val_hbm, out_hbm,
             acc, srcb, dstb, valb, rows, gsem, ssem):
    cid = lax.axis_index("c")
    sid = lax.axis_index("s")
    wid = cid * NS + sid

    def _zrow(i, carry):
        for q in range(D // 16):
            rows[0, i, pl.ds(q * 16, 16)] = jnp.zeros((16,), jnp.float32)
        return carry
    lax.fori_loop(0, CHUNK, _zrow, 0)
    for k in range(ROWS_PER_TILE // CHUNK):
        pltpu.sync_copy(rows.at[0],
                        acc.at[pl.ds(sid * ROWS_PER_TILE + k * CHUNK, CHUNK)])

    plsc.subcore_barrier()

    def _group(og, carry):
        pltpu.sync_copy(src_hbm.at[wid, og], srcb)
        pltpu.sync_copy(dst_hbm.at[wid, og], dstb)
        pltpu.sync_copy(val_hbm.at[wid, og], valb)
        pltpu.async_copy(x_hbm.at[srcb.at[0]], rows.at[0], gsem.at[0])

        def _chunk(j, carry2):
            p = j % 3
            nxt = (j + 1) % 3
            @pl.when(j >= 2)
            def _wait_prev():
                pltpu.make_async_copy(rows.at[nxt], acc.at[dstb.at[j - 2]],
                                      ssem.at[nxt]).wait()
            @pl.when(j + 1 < GCHUNK)
            def _start_next():
                pltpu.async_copy(x_hbm.at[srcb.at[j + 1]], rows.at[nxt],
                                 gsem.at[nxt])
            pltpu.make_async_copy(x_hbm.at[srcb.at[j]], rows.at[p],
                                  gsem.at[p]).wait()
            for g in range(CHUNK // 16):
                vv = valb[j, pl.ds(g * 16, 16)]
                for l in range(16):
                    e = g * 16 + l
                    vb = _bcast_lane(vv, l)
                    for q in range(D // 16):
                        rows[p, e, pl.ds(q * 16, 16)] = (
                            rows[p, e, pl.ds(q * 16, 16)] * vb)
            pltpu.async_copy(rows.at[p], acc.at[dstb.at[j]], ssem.at[p],
                             add=True)
            return carry2
        lax.fori_loop(0, GCHUNK, _chunk, 0)
        for jj in (GCHUNK - 2, GCHUNK - 1):
            pp = jj % 3
            pltpu.make_async_copy(rows.at[pp], acc.at[dstb.at[jj]],
                                  ssem.at[pp]).wait()
        return carry
    lax.fori_loop(0, GROUPS, _group, 0)

    plsc.subcore_barrier()
    for k in range(ROWS_PER_TILE // ZROWS):
        r0 = sid * ROWS_PER_TILE + k * ZROWS
        pltpu.sync_copy(acc.at[pl.ds(r0, ZROWS)], out_hbm.at[cid, pl.ds(r0, ZROWS)])


@functools.cache
def _sc_agg():
    return pl.kernel(
        _sc_body,
        out_type=jax.ShapeDtypeStruct((NC, N_ACC, D), jnp.float32),
        mesh=plsc.VectorSubcoreMesh(core_axis_name="c", subcore_axis_name="s",
                                    num_cores=NC, num_subcores=NS),
        scratch_types=[
            pltpu.VMEM_SHARED((N_ACC, D), jnp.float32),
            pltpu.VMEM((GCHUNK, CHUNK), jnp.int32),
            pltpu.VMEM((GCHUNK, CHUNK), jnp.int32),
            pltpu.VMEM((GCHUNK, CHUNK), jnp.float32),
            pltpu.VMEM((3, CHUNK, D), jnp.float32),
            pltpu.SemaphoreType.DMA((3,)),
            pltpu.SemaphoreType.DMA((3,)),
        ],
    )


def kernel(input, adj_edge_index, adj_values, W, b, self_c):
    c2 = self_c.reshape(1, 1)
    b2 = b.reshape(1, D)
    h = _dense_stage(input, W, b2, c2)
    pad = E_PAD - E
    src = jnp.pad(adj_edge_index[1], (0, pad)).reshape(NW, GROUPS, GCHUNK, CHUNK)
    dst = jnp.pad(adj_edge_index[0], (0, pad)).reshape(NW, GROUPS, GCHUNK, CHUNK)
    val = jnp.pad(adj_values, (0, pad)).reshape(NW, GROUPS, GCHUNK, CHUNK)
    parts = _sc_agg()(h, src, dst, val)
    return _final_stage(parts, c2)


# speedup vs baseline: 2.8850x; 1.0718x over previous
import functools

import jax
import jax.numpy as jnp
from jax import lax
from jax.experimental import pallas as pl
from jax.experimental.pallas import tpu as pltpu
from jax.experimental.pallas import tpu_sc as plsc

EPS = 1e-15
N = 10000
E = 320000
D = 128

NC = 2
NS = 16
NW = NC * NS
CHUNK = 80
GCHUNK = 25
GROUPS = 5
EDGES_PER_TILE = GROUPS * GCHUNK * CHUNK
N_ACC = 10240
ROWS_PER_TILE = N_ACC // NS
ZROWS = 128

BLK = 2000


def _artanh(x):
    x = jnp.clip(x, -1.0 + 1e-5, 1.0 - 1e-5)
    return 0.5 * (jnp.log1p(x) - jnp.log1p(-x))


def _rownorm(x):
    return jnp.clip(jnp.sqrt(jnp.sum(x * x, axis=-1, keepdims=True)), EPS, None)


def _dense_body(x_ref, w_ref, b_ref, c_ref, o_ref):
    c = c_ref[...]
    sqrt_c = jnp.sqrt(c)
    x = x_ref[...]
    x_norm = _rownorm(x)
    mx = jnp.dot(x, w_ref[...], preferred_element_type=jnp.float32)
    mx_norm = _rownorm(mx)
    res = jnp.tanh(mx_norm / x_norm * _artanh(sqrt_c * x_norm)) * mx / (mx_norm * sqrt_c)
    zero_mask = jnp.max(jnp.abs(mx), axis=-1, keepdims=True) == 0.0
    out = jnp.where(zero_mask, 0.0, res)

    bias = b_ref[...]
    bn = _rownorm(bias)
    hb = jnp.tanh(sqrt_c * bn) * bias / (sqrt_c * bn)
    hbn = _rownorm(hb)
    maxnorm = (1.0 - 4e-3) / sqrt_c
    hb = jnp.where(hbn > maxnorm, hb / hbn * maxnorm, hb)

    x2 = jnp.sum(out * out, axis=-1, keepdims=True)
    y2 = jnp.sum(hb * hb, axis=-1, keepdims=True)
    xy = jnp.sum(out * hb, axis=-1, keepdims=True)
    num = (1.0 + 2.0 * c * xy + c * y2) * out + (1.0 - c * x2) * hb
    den = 1.0 + 2.0 * c * xy + c * c * x2 * y2
    oa = num / jnp.clip(den, EPS, None)

    on = _rownorm(oa)
    o_ref[...] = _artanh(sqrt_c * on) * oa / (sqrt_c * on)


def _dense_stage(x, w, b2, c2):
    return pl.pallas_call(
        _dense_body,
        grid=(N // BLK,),
        in_specs=[
            pl.BlockSpec((BLK, D), lambda i: (i, 0)),
            pl.BlockSpec((D, D), lambda i: (0, 0)),
            pl.BlockSpec((1, D), lambda i: (0, 0)),
            pl.BlockSpec((1, 1), lambda i: (0, 0)),
        ],
        out_specs=pl.BlockSpec((BLK, D), lambda i: (i, 0)),
        out_shape=jax.ShapeDtypeStruct((N, D), jnp.float32),
    )(x, w, b2, c2)


def _final_body(p_ref, c_ref, o_ref):
    c = c_ref[...]
    sqrt_c = jnp.sqrt(c)
    s = p_ref[0] + p_ref[1]
    un = _rownorm(s)
    em = jnp.tanh(sqrt_c * un) * s / (sqrt_c * un)
    en = _rownorm(em)
    maxnorm = (1.0 - 4e-3) / sqrt_c
    o_ref[...] = jnp.where(en > maxnorm, em / en * maxnorm, em)


FBLK = 1000


def _final_stage(parts, c2):
    return pl.pallas_call(
        _final_body,
        grid=(N // FBLK,),
        in_specs=[
            pl.BlockSpec((NC, FBLK, D), lambda i: (0, i, 0)),
            pl.BlockSpec((1, 1), lambda i: (0, 0)),
        ],
        out_specs=pl.BlockSpec((FBLK, D), lambda i: (i, 0)),
        out_shape=jax.ShapeDtypeStruct((N, D), jnp.float32),
    )(parts, c2)


def _bcast_lane(vec, lane):
    idx = jnp.full((16, 1), lane, jnp.int32)
    dnums = lax.GatherDimensionNumbers(
        offset_dims=(), collapsed_slice_dims=(0,), start_index_map=(0,))
    return lax.gather(vec, idx, dnums, (1,),
                      mode=lax.GatherScatterMode.PROMISE_IN_BOUNDS)


GEDGES = GCHUNK * CHUNK


def _sc_body(ei_hbm, val_hbm, x_hbm, out_hbm,
             acc, srcb, dstb, valb, dst2, rows, gsem, ssem, zsem):
    cid = lax.axis_index("c")
    sid = lax.axis_index("s")
    wid = cid * NS + sid
    ebase = wid * EDGES_PER_TILE

    def _zrow(i, carry):
        for q in range(D // 16):
            rows[0, i, pl.ds(q * 16, 16)] = jnp.zeros((16,), jnp.float32)
        return carry
    lax.fori_loop(0, CHUNK, _zrow, 0)
    nz = ROWS_PER_TILE // CHUNK
    for k in range(nz):
        pltpu.async_copy(rows.at[0],
                         acc.at[pl.ds(sid * ROWS_PER_TILE + k * CHUNK, CHUNK)],
                         zsem)
    for k in range(nz):
        pltpu.make_async_copy(
            rows.at[0],
            acc.at[pl.ds(sid * ROWS_PER_TILE + k * CHUNK, CHUNK)], zsem).wait()

    plsc.subcore_barrier()

    def _group(og, carry):
        gb = ebase + og * GEDGES
        pltpu.sync_copy(ei_hbm.at[pl.ds(gb, GEDGES)], dstb)
        pltpu.sync_copy(ei_hbm.at[pl.ds(E + gb, GEDGES)], srcb)
        pltpu.sync_copy(val_hbm.at[pl.ds(gb, GEDGES)], valb)
        pltpu.async_copy(x_hbm.at[srcb.at[pl.ds(0, CHUNK)]], rows.at[0],
                         gsem.at[0])

        def _chunk(j, carry2):
            p = j % 3
            nxt = (j + 1) % 3
            @pl.when(j >= 2)
            def _wait_prev():
                pltpu.make_async_copy(rows.at[nxt], acc.at[dst2.at[nxt]],
                                      ssem.at[nxt]).wait()
            @pl.when(j + 1 < GCHUNK)
            def _start_next():
                pltpu.async_copy(x_hbm.at[srcb.at[pl.ds((j + 1) * CHUNK, CHUNK)]],
                                 rows.at[nxt], gsem.at[nxt])
            pltpu.make_async_copy(x_hbm.at[srcb.at[pl.ds(j * CHUNK, CHUNK)]],
                                  rows.at[p], gsem.at[p]).wait()
            for k in range(CHUNK // 16):
                dst2[p, pl.ds(k * 16, 16)] = dstb[pl.ds(j * CHUNK + k * 16, 16)]
            for g in range(CHUNK // 16):
                vv = valb[pl.ds(j * CHUNK + g * 16, 16)]
                for l in range(16):
                    e = g * 16 + l
                    vb = _bcast_lane(vv, l)
                    for q in range(D // 16):
                        rows[p, e, pl.ds(q * 16, 16)] = (
                            rows[p, e, pl.ds(q * 16, 16)] * vb)
            pltpu.async_copy(rows.at[p], acc.at[dst2.at[p]], ssem.at[p],
                             add=True)
            return carry2
        lax.fori_loop(0, GCHUNK, _chunk, 0)
        for jj in (GCHUNK - 2, GCHUNK - 1):
            pp = jj % 3
            pltpu.make_async_copy(rows.at[pp], acc.at[dst2.at[pp]],
                                  ssem.at[pp]).wait()
        return carry
    lax.fori_loop(0, GROUPS, _group, 0)

    plsc.subcore_barrier()
    for k in range(ROWS_PER_TILE // ZROWS):
        r0 = sid * ROWS_PER_TILE + k * ZROWS
        pltpu.async_copy(acc.at[pl.ds(r0, ZROWS)],
                         out_hbm.at[cid, pl.ds(r0, ZROWS)], zsem)
    for k in range(ROWS_PER_TILE // ZROWS):
        r0 = sid * ROWS_PER_TILE + k * ZROWS
        pltpu.make_async_copy(acc.at[pl.ds(r0, ZROWS)],
                              out_hbm.at[cid, pl.ds(r0, ZROWS)], zsem).wait()


@functools.cache
def _sc_agg():
    return pl.kernel(
        _sc_body,
        out_type=jax.ShapeDtypeStruct((NC, N_ACC, D), jnp.float32),
        mesh=plsc.VectorSubcoreMesh(core_axis_name="c", subcore_axis_name="s",
                                    num_cores=NC, num_subcores=NS),
        scratch_types=[
            pltpu.VMEM_SHARED((N_ACC, D), jnp.float32),
            pltpu.VMEM((GEDGES,), jnp.int32),
            pltpu.VMEM((GEDGES,), jnp.int32),
            pltpu.VMEM((GEDGES,), jnp.float32),
            pltpu.VMEM((3, CHUNK), jnp.int32),
            pltpu.VMEM((3, CHUNK, D), jnp.float32),
            pltpu.SemaphoreType.DMA((3,)),
            pltpu.SemaphoreType.DMA((3,)),
            pltpu.SemaphoreType.DMA,
        ],
    )


def kernel(input, adj_edge_index, adj_values, W, b, self_c):
    c2 = self_c.reshape(1, 1)
    b2 = b.reshape(1, D)
    h = _dense_stage(input, W, b2, c2)
    ei_flat = adj_edge_index.reshape(2 * E)
    parts = _sc_agg()(ei_flat, adj_values, h)
    return _final_stage(parts, c2)
